# baseline (speedup 1.0000x reference)
import jax
import jax.numpy as jnp
from jax.experimental import pallas as pl
from jax.experimental.pallas import tpu as pltpu


_BM = 1024
_BN = 512


def _matmul_kernel(x_ref, w_ref, b_ref, o_ref, xb_ref):
    @pl.when(pl.program_id(1) == 0)
    def _cast_x():
        xb_ref[...] = x_ref[...].astype(jnp.bfloat16)

    acc = jax.lax.dot_general(
        xb_ref[...], w_ref[...],
        dimension_numbers=(((1,), (0,)), ((), ())),
        preferred_element_type=jnp.float32,
    )
    o_ref[...] = acc + b_ref[...]


def kernel(input, W, bias):
    B, S, K = input.shape
    N = W.shape[0]
    M = B * S
    x = input.reshape(M, K)
    wt = W.T.astype(jnp.bfloat16)
    b = bias.reshape(1, N)

    grid = (M // _BM, N // _BN)

    out = pl.pallas_call(
        _matmul_kernel,
        grid=grid,
        in_specs=[
            pl.BlockSpec((_BM, K), lambda i, j: (i, 0)),
            pl.BlockSpec((K, _BN), lambda i, j: (0, j)),
            pl.BlockSpec((1, _BN), lambda i, j: (0, j)),
        ],
        out_specs=pl.BlockSpec((_BM, _BN), lambda i, j: (i, j)),
        out_shape=jax.ShapeDtypeStruct((M, N), jnp.float32),
        scratch_shapes=[pltpu.VMEM((_BM, K), jnp.bfloat16)],
    )(x, wt, b)
    return out.reshape(B, S, N)


# speedup vs baseline: 1.0244x; 1.0244x over previous
import jax
import jax.numpy as jnp
from jax.experimental import pallas as pl
from jax.experimental.pallas import tpu as pltpu


_BM = 1024
_BN = 512


def _matmul_kernel(x_ref, w_ref, b_ref, o_ref):
    acc = jax.lax.dot_general(
        x_ref[...].astype(jnp.bfloat16), w_ref[...],
        dimension_numbers=(((1,), (0,)), ((), ())),
        preferred_element_type=jnp.float32,
    )
    o_ref[...] = acc + b_ref[...]


def kernel(input, W, bias):
    B, S, K = input.shape
    N = W.shape[0]
    M = B * S
    x = input.reshape(M, K)
    wt = W.T.astype(jnp.bfloat16)
    b = bias.reshape(1, N)

    grid = (M // _BM, N // _BN)

    out = pl.pallas_call(
        _matmul_kernel,
        grid=grid,
        in_specs=[
            pl.BlockSpec((_BM, K), lambda i, j: (i, 0)),
            pl.BlockSpec((K, _BN), lambda i, j: (0, j)),
            pl.BlockSpec((1, _BN), lambda i, j: (0, j)),
        ],
        out_specs=pl.BlockSpec((_BM, _BN), lambda i, j: (i, j)),
        out_shape=jax.ShapeDtypeStruct((M, N), jnp.float32),
    )(x, wt, b)
    return out.reshape(B, S, N)


# speedup vs baseline: 1.0723x; 1.0467x over previous
import jax
import jax.numpy as jnp
from jax.experimental import pallas as pl
from jax.experimental.pallas import tpu as pltpu


_BM = 256
_BN = 1024


def _matmul_kernel(x_ref, w_ref, b_ref, o_ref, wb_ref):
    @pl.when(pl.program_id(1) == 0)
    def _cast_w():
        wb_ref[...] = w_ref[...].astype(jnp.bfloat16)

    acc = jax.lax.dot_general(
        x_ref[...].astype(jnp.bfloat16), wb_ref[...],
        dimension_numbers=(((1,), (1,)), ((), ())),
        preferred_element_type=jnp.float32,
    )
    o_ref[...] = acc + b_ref[...]


def kernel(input, W, bias):
    B, S, K = input.shape
    N = W.shape[0]
    M = B * S
    x = input.reshape(M, K)
    b = bias.reshape(1, N)

    grid = (N // _BN, M // _BM)

    out = pl.pallas_call(
        _matmul_kernel,
        grid=grid,
        in_specs=[
            pl.BlockSpec((_BM, K), lambda j, i: (i, 0)),
            pl.BlockSpec((_BN, K), lambda j, i: (j, 0)),
            pl.BlockSpec((1, _BN), lambda j, i: (0, j)),
        ],
        out_specs=pl.BlockSpec((_BM, _BN), lambda j, i: (i, j)),
        out_shape=jax.ShapeDtypeStruct((M, N), jnp.float32),
        scratch_shapes=[pltpu.VMEM((_BN, K), jnp.bfloat16)],
    )(x, W, b)
    return out.reshape(B, S, N)


# speedup vs baseline: 1.1955x; 1.1149x over previous
import jax
import jax.numpy as jnp
from jax.experimental import pallas as pl


_BM = 512
_BN = 1024


def _matmul_kernel(x_ref, w_ref, b_ref, o_ref):
    acc = jax.lax.dot_general(
        x_ref[...], w_ref[...],
        dimension_numbers=(((1,), (1,)), ((), ())),
        precision=jax.lax.Precision.DEFAULT,
        preferred_element_type=jnp.float32,
    )
    o_ref[...] = acc + b_ref[...]


def kernel(input, W, bias):
    B, S, K = input.shape
    N = W.shape[0]
    M = B * S
    x = input.reshape(M, K)
    b = bias.reshape(1, N)

    grid = (N // _BN, M // _BM)

    out = pl.pallas_call(
        _matmul_kernel,
        grid=grid,
        in_specs=[
            pl.BlockSpec((_BM, K), lambda j, i: (i, 0)),
            pl.BlockSpec((_BN, K), lambda j, i: (j, 0)),
            pl.BlockSpec((1, _BN), lambda j, i: (0, j)),
        ],
        out_specs=pl.BlockSpec((_BM, _BN), lambda j, i: (i, j)),
        out_shape=jax.ShapeDtypeStruct((M, N), jnp.float32),
    )(x, W, b)
    return out.reshape(B, S, N)
